# baseline (speedup 1.0000x reference)
import functools

import jax
import jax.numpy as jnp
from jax import lax
from jax.experimental import pallas as pl
from jax.experimental.pallas import tpu as pltpu
from jax.experimental.pallas import tpu_sc as plsc

N = 10000
E = 320000
D = 128
R = 200

NC = 2
NS = 16
NW = NC * NS
NPAD = 10240
ROWS_PER_TILE = NPAD // NS
K = 40
CHUNKS = 256
EDGES_PER_TILE = CHUNKS * K
EPAD = NW * EDGES_PER_TILE
NBUF = 4
NIDX = 8

BLK = 2000


def _dense_body(x_ref, ph_ref, wn_ref, wl_ref, ws_ref, b_ref,
                xw_ref, loop_ref, gate_ref):
    xb = x_ref[...]
    xw_ref[...] = jnp.dot(xb, wn_ref[...], preferred_element_type=jnp.float32)
    loop_ref[...] = jnp.dot(xb, wl_ref[...], preferred_element_type=jnp.float32)
    z = jnp.dot(ph_ref[...], ws_ref[...], preferred_element_type=jnp.float32)
    gate_ref[...] = jax.nn.sigmoid(z + b_ref[...])


def _tc_dense(x, prev_h, wn, wl, ws, b2d):
    grid = (N // BLK,)
    row_spec = pl.BlockSpec((BLK, D), lambda i: (i, 0))
    w_spec = pl.BlockSpec((D, D), lambda i: (0, 0))
    b_spec = pl.BlockSpec((1, D), lambda i: (0, 0))
    out_sds = jax.ShapeDtypeStruct((N, D), jnp.float32)
    return pl.pallas_call(
        _dense_body,
        grid=grid,
        in_specs=[row_spec, row_spec, w_spec, w_spec, w_spec, b_spec],
        out_specs=[row_spec, row_spec, row_spec],
        out_shape=[out_sds, out_sds, out_sds],
    )(x, prev_h, wn, wl, ws, b2d)


def _embw_body(e_ref, w_ref, o_ref):
    o_ref[...] = jnp.dot(e_ref[...], w_ref[...],
                         preferred_element_type=jnp.float32)


def _tc_embw(emb_rel, wn):
    return pl.pallas_call(
        _embw_body,
        out_shape=jax.ShapeDtypeStruct((R, D), jnp.float32),
    )(emb_rel, wn)


def _sc_edge_body(xw_hbm, embw_hbm, srcI, typI, dstI, zeros_hbm,
                  out_hbm, acc, *bufs):
    c = lax.axis_index("c")
    s = lax.axis_index("s")
    wid = c * NS + s
    srcb = bufs[0:NIDX]
    typb = bufs[NIDX:2 * NIDX]
    dstb = bufs[2 * NIDX:3 * NIDX]
    xbufs = bufs[3 * NIDX:3 * NIDX + NBUF]
    rbufs = bufs[3 * NIDX + NBUF:3 * NIDX + 2 * NBUF]
    semoff = 3 * NIDX + 2 * NBUF
    isem = bufs[semoff:semoff + NIDX]
    gsem = bufs[semoff + NIDX:semoff + NIDX + NBUF]
    ssem = bufs[semoff + NIDX + NBUF:semoff + NIDX + 2 * NBUF]

    pltpu.sync_copy(zeros_hbm, acc.at[pl.ds(s * ROWS_PER_TILE, ROWS_PER_TILE)])

    def i_issue(i, q):
        pltpu.async_copy(srcI.at[wid, i], srcb[q], isem[q])
        pltpu.async_copy(typI.at[wid, i], typb[q], isem[q])
        pltpu.async_copy(dstI.at[wid, i], dstb[q], isem[q])

    def i_wait(q):
        pltpu.make_async_copy(srcI.at[0, 0], srcb[q], isem[q]).wait()
        pltpu.make_async_copy(typI.at[0, 0], typb[q], isem[q]).wait()
        pltpu.make_async_copy(dstI.at[0, 0], dstb[q], isem[q]).wait()

    def g_issue(q, b):
        pltpu.async_copy(xw_hbm.at[srcb[q]], xbufs[b], gsem[b])
        pltpu.async_copy(embw_hbm.at[typb[q]], rbufs[b], gsem[b])

    def g_wait(b):
        pltpu.make_async_copy(xw_hbm.at[srcb[0]], xbufs[b], gsem[b]).wait()
        pltpu.make_async_copy(embw_hbm.at[typb[0]], rbufs[b], gsem[b]).wait()

    def s_issue(q, b):
        pltpu.async_copy(xbufs[b], acc.at[dstb[q]], ssem[b], add=True)
        pltpu.async_copy(rbufs[b], acc.at[dstb[q]], ssem[b], add=True)

    def s_wait(b):
        pltpu.make_async_copy(xbufs[b], acc.at[dstb[0]], ssem[b]).wait()
        pltpu.make_async_copy(rbufs[b], acc.at[dstb[0]], ssem[b]).wait()

    for i in range(6):
        i_issue(i, i)
    i_wait(0)
    g_issue(0, 0)
    i_wait(1)
    g_issue(1, 1)
    plsc.subcore_barrier()

    def outer(jo, carry):
        for u in range(NIDX):
            i = NIDX * jo + u
            b = u % NBUF
            bp = (b + 2) % NBUF
            q2 = (u + 2) % NIDX
            q6 = (u + 6) % NIDX
            g_wait(b)
            s_issue(u, b)

            @pl.when(i >= 2)
            def _():
                s_wait(bp)

            @pl.when(i <= CHUNKS - 3)
            def _():
                i_wait(q2)
                g_issue(q2, bp)

            @pl.when(i <= CHUNKS - 7)
            def _():
                i_issue(i + 6, q6)

        return carry

    lax.fori_loop(0, CHUNKS // NIDX, outer, 0)
    s_wait((CHUNKS - 2) % NBUF)
    s_wait((CHUNKS - 1) % NBUF)
    plsc.subcore_barrier()
    pltpu.sync_copy(acc.at[pl.ds(s * ROWS_PER_TILE, ROWS_PER_TILE)],
                    out_hbm.at[c, pl.ds(s * ROWS_PER_TILE, ROWS_PER_TILE)])


def _sc_edges(xw, embw, srcI, typI, dstI, zeros):
    mesh = plsc.VectorSubcoreMesh(core_axis_name="c", subcore_axis_name="s")
    fn = functools.partial(
        pl.kernel,
        mesh=mesh,
        out_type=jax.ShapeDtypeStruct((NC, NPAD, D), jnp.float32),
        scratch_types=[pltpu.VMEM_SHARED((NPAD, D), jnp.float32)]
        + [pltpu.VMEM((K,), jnp.int32)] * (3 * NIDX)
        + [pltpu.VMEM((K, D), jnp.float32)] * (2 * NBUF)
        + [pltpu.SemaphoreType.DMA] * (NIDX + 2 * NBUF),
    )(_sc_edge_body)
    return fn(xw, embw, srcI, typI, dstI, zeros)


def _final_body(p_ref, norm_ref, loop_ref, gate_ref, prev_ref, o_ref):
    agg = p_ref[0] + p_ref[1]
    h = agg * norm_ref[...] + loop_ref[...]
    g = gate_ref[...]
    o_ref[...] = g * h + (1.0 - g) * prev_ref[...]


def _tc_final(partials, norm, loop_m, gate, prev_h):
    grid = (N // BLK,)
    row_spec = pl.BlockSpec((BLK, D), lambda i: (i, 0))
    p_spec = pl.BlockSpec((NC, BLK, D), lambda i: (0, i, 0))
    n_spec = pl.BlockSpec((BLK, 1), lambda i: (i, 0))
    return pl.pallas_call(
        _final_body,
        grid=grid,
        in_specs=[p_spec, n_spec, row_spec, row_spec, row_spec],
        out_specs=pl.BlockSpec((BLK, D), lambda i: (i, 0)),
        out_shape=jax.ShapeDtypeStruct((N, D), jnp.float32),
    )(partials, norm, loop_m, gate, prev_h)


def kernel(x, edge_index, edge_type, norm, prev_h, emb_rel,
           weight_neighbor, loop_weight, skip_connect_weight,
           skip_connect_bias):
    src = edge_index[0]
    dst = edge_index[1]
    b2d = skip_connect_bias.reshape(1, D)
    xw, loop_m, gate = _tc_dense(x, prev_h, weight_neighbor, loop_weight,
                                 skip_connect_weight, b2d)
    embw = _tc_embw(emb_rel, weight_neighbor)
    zeros = jnp.zeros((ROWS_PER_TILE, D), jnp.float32)
    pad = EPAD - E
    srcI = jnp.concatenate([src, jnp.zeros((pad,), jnp.int32)]
                           ).reshape(NW, CHUNKS, K)
    typI = jnp.concatenate([edge_type, jnp.zeros((pad,), jnp.int32)]
                           ).reshape(NW, CHUNKS, K)
    dstI = jnp.concatenate([dst, jnp.full((pad,), N, jnp.int32)]
                           ).reshape(NW, CHUNKS, K)
    partials = _sc_edges(xw, embw, srcI, typI, dstI, zeros)
    return _tc_final(partials[:, :N], norm, loop_m, gate, prev_h)


# speedup vs baseline: 1.0696x; 1.0696x over previous
import functools

import jax
import jax.numpy as jnp
from jax import lax
from jax.experimental import pallas as pl
from jax.experimental.pallas import tpu as pltpu
from jax.experimental.pallas import tpu_sc as plsc

N = 10000
E = 320000
D = 128
R = 200

NC = 2
NS = 16
NW = NC * NS
NPAD = 10240
ROWS_PER_TILE = NPAD // NS
K = 40
CHUNKS = 256
EDGES_PER_TILE = CHUNKS * K
EPAD = NW * EDGES_PER_TILE
NBUF = 4
NIDX = 8

BLK = 2000


def _dense_body(x_ref, ph_ref, wn_ref, wl_ref, ws_ref, b_ref,
                xw_ref, loop_ref, gate_ref):
    xb = x_ref[...]
    xw_ref[...] = jnp.dot(xb, wn_ref[...], preferred_element_type=jnp.float32)
    loop_ref[...] = jnp.dot(xb, wl_ref[...], preferred_element_type=jnp.float32)
    z = jnp.dot(ph_ref[...], ws_ref[...], preferred_element_type=jnp.float32)
    gate_ref[...] = jax.nn.sigmoid(z + b_ref[...])


def _tc_dense(x, prev_h, wn, wl, ws, b2d):
    grid = (N // BLK,)
    row_spec = pl.BlockSpec((BLK, D), lambda i: (i, 0))
    w_spec = pl.BlockSpec((D, D), lambda i: (0, 0))
    b_spec = pl.BlockSpec((1, D), lambda i: (0, 0))
    out_sds = jax.ShapeDtypeStruct((N, D), jnp.float32)
    return pl.pallas_call(
        _dense_body,
        grid=grid,
        in_specs=[row_spec, row_spec, w_spec, w_spec, w_spec, b_spec],
        out_specs=[row_spec, row_spec, row_spec],
        out_shape=[out_sds, out_sds, out_sds],
    )(x, prev_h, wn, wl, ws, b2d)


def _embw_body(e_ref, w_ref, o_ref):
    o_ref[...] = jnp.dot(e_ref[...], w_ref[...],
                         preferred_element_type=jnp.float32)


def _tc_embw(emb_rel, wn):
    return pl.pallas_call(
        _embw_body,
        out_shape=jax.ShapeDtypeStruct((R, D), jnp.float32),
    )(emb_rel, wn)


def _sc_edge_body(xw_hbm, embw_hbm, srcI, typI, dstI, zeros_hbm,
                  out_hbm, acc, embw_sp, *bufs):
    c = lax.axis_index("c")
    s = lax.axis_index("s")
    wid = c * NS + s
    srcb = bufs[0:NIDX]
    typb = bufs[NIDX:2 * NIDX]
    dstb = bufs[2 * NIDX:3 * NIDX]
    xbufs = bufs[3 * NIDX:3 * NIDX + NBUF]
    rbufs = bufs[3 * NIDX + NBUF:3 * NIDX + 2 * NBUF]
    semoff = 3 * NIDX + 2 * NBUF
    isem = bufs[semoff:semoff + NIDX]
    gsem = bufs[semoff + NIDX:semoff + NIDX + NBUF]
    ssem = bufs[semoff + NIDX + NBUF:semoff + NIDX + 2 * NBUF]
    rsem = bufs[semoff + NIDX + 2 * NBUF:semoff + NIDX + 3 * NBUF]

    pltpu.sync_copy(zeros_hbm, acc.at[pl.ds(s * ROWS_PER_TILE, ROWS_PER_TILE)])

    @pl.when(s == 0)
    def _():
        pltpu.sync_copy(embw_hbm, embw_sp)

    def i_issue(i, q):
        pltpu.async_copy(srcI.at[wid, i], srcb[q], isem[q])
        pltpu.async_copy(typI.at[wid, i], typb[q], isem[q])
        pltpu.async_copy(dstI.at[wid, i], dstb[q], isem[q])

    def i_wait(q):
        pltpu.make_async_copy(srcI.at[0, 0], srcb[q], isem[q]).wait()
        pltpu.make_async_copy(typI.at[0, 0], typb[q], isem[q]).wait()
        pltpu.make_async_copy(dstI.at[0, 0], dstb[q], isem[q]).wait()

    def g_issue(q, b):
        pltpu.async_copy(xw_hbm.at[srcb[q]], xbufs[b], gsem[b])
        pltpu.async_copy(embw_sp.at[typb[q]], rbufs[b], rsem[b])

    def g_wait(b):
        pltpu.make_async_copy(xw_hbm.at[srcb[0]], xbufs[b], gsem[b]).wait()
        pltpu.make_async_copy(embw_sp.at[typb[0]], rbufs[b], rsem[b]).wait()

    def s_issue(q, b):
        pltpu.async_copy(xbufs[b], acc.at[dstb[q]], ssem[b], add=True)
        pltpu.async_copy(rbufs[b], acc.at[dstb[q]], ssem[b], add=True)

    def s_wait(b):
        pltpu.make_async_copy(xbufs[b], acc.at[dstb[0]], ssem[b]).wait()
        pltpu.make_async_copy(rbufs[b], acc.at[dstb[0]], ssem[b]).wait()

    for i in range(6):
        i_issue(i, i)
    plsc.subcore_barrier()
    i_wait(0)
    g_issue(0, 0)
    i_wait(1)
    g_issue(1, 1)

    def outer(jo, carry):
        for u in range(NIDX):
            i = NIDX * jo + u
            b = u % NBUF
            bp = (b + 2) % NBUF
            q2 = (u + 2) % NIDX
            q6 = (u + 6) % NIDX
            g_wait(b)
            s_issue(u, b)

            @pl.when(i >= 2)
            def _():
                s_wait(bp)

            @pl.when(i <= CHUNKS - 3)
            def _():
                i_wait(q2)
                g_issue(q2, bp)

            @pl.when(i <= CHUNKS - 7)
            def _():
                i_issue(i + 6, q6)

        return carry

    lax.fori_loop(0, CHUNKS // NIDX, outer, 0)
    s_wait((CHUNKS - 2) % NBUF)
    s_wait((CHUNKS - 1) % NBUF)
    plsc.subcore_barrier()
    pltpu.sync_copy(acc.at[pl.ds(s * ROWS_PER_TILE, ROWS_PER_TILE)],
                    out_hbm.at[c, pl.ds(s * ROWS_PER_TILE, ROWS_PER_TILE)])


def _sc_edges(xw, embw, srcI, typI, dstI, zeros):
    mesh = plsc.VectorSubcoreMesh(core_axis_name="c", subcore_axis_name="s")
    fn = functools.partial(
        pl.kernel,
        mesh=mesh,
        out_type=jax.ShapeDtypeStruct((NC, NPAD, D), jnp.float32),
        scratch_types=[pltpu.VMEM_SHARED((NPAD, D), jnp.float32),
                       pltpu.VMEM_SHARED((R, D), jnp.float32)]
        + [pltpu.VMEM((K,), jnp.int32)] * (3 * NIDX)
        + [pltpu.VMEM((K, D), jnp.float32)] * (2 * NBUF)
        + [pltpu.SemaphoreType.DMA] * (NIDX + 3 * NBUF),
    )(_sc_edge_body)
    return fn(xw, embw, srcI, typI, dstI, zeros)


def _final_body(p_ref, norm_ref, loop_ref, gate_ref, prev_ref, o_ref):
    agg = p_ref[0] + p_ref[1]
    h = agg * norm_ref[...] + loop_ref[...]
    g = gate_ref[...]
    o_ref[...] = g * h + (1.0 - g) * prev_ref[...]


def _tc_final(partials, norm, loop_m, gate, prev_h):
    grid = (N // BLK,)
    row_spec = pl.BlockSpec((BLK, D), lambda i: (i, 0))
    p_spec = pl.BlockSpec((NC, BLK, D), lambda i: (0, i, 0))
    n_spec = pl.BlockSpec((BLK, 1), lambda i: (i, 0))
    return pl.pallas_call(
        _final_body,
        grid=grid,
        in_specs=[p_spec, n_spec, row_spec, row_spec, row_spec],
        out_specs=pl.BlockSpec((BLK, D), lambda i: (i, 0)),
        out_shape=jax.ShapeDtypeStruct((N, D), jnp.float32),
    )(partials, norm, loop_m, gate, prev_h)


def kernel(x, edge_index, edge_type, norm, prev_h, emb_rel,
           weight_neighbor, loop_weight, skip_connect_weight,
           skip_connect_bias):
    src = edge_index[0]
    dst = edge_index[1]
    b2d = skip_connect_bias.reshape(1, D)
    xw, loop_m, gate = _tc_dense(x, prev_h, weight_neighbor, loop_weight,
                                 skip_connect_weight, b2d)
    embw = _tc_embw(emb_rel, weight_neighbor)
    zeros = jnp.zeros((ROWS_PER_TILE, D), jnp.float32)
    pad = EPAD - E
    srcI = jnp.concatenate([src, jnp.zeros((pad,), jnp.int32)]
                           ).reshape(NW, CHUNKS, K)
    typI = jnp.concatenate([edge_type, jnp.zeros((pad,), jnp.int32)]
                           ).reshape(NW, CHUNKS, K)
    dstI = jnp.concatenate([dst, jnp.full((pad,), N, jnp.int32)]
                           ).reshape(NW, CHUNKS, K)
    partials = _sc_edges(xw, embw, srcI, typI, dstI, zeros)
    return _tc_final(partials[:, :N], norm, loop_m, gate, prev_h)


# speedup vs baseline: 1.0814x; 1.0111x over previous
import functools

import jax
import jax.numpy as jnp
from jax import lax
from jax.experimental import pallas as pl
from jax.experimental.pallas import tpu as pltpu
from jax.experimental.pallas import tpu_sc as plsc

N = 10000
E = 320000
D = 128
R = 200

NC = 2
NS = 16
NW = NC * NS
NPAD = 10240
ROWS_PER_TILE = NPAD // NS
K = 40
KH = K // 2
CHUNKS = 256
EDGES_PER_TILE = CHUNKS * K
EPAD = NW * EDGES_PER_TILE
NBUF = 4
NIDX = 8

BLK = 2000


def _dense_body(x_ref, ph_ref, wn_ref, wl_ref, ws_ref, b_ref,
                xw_ref, loop_ref, gate_ref):
    xb = x_ref[...]
    xw_ref[...] = jnp.dot(xb, wn_ref[...], preferred_element_type=jnp.float32)
    loop_ref[...] = jnp.dot(xb, wl_ref[...], preferred_element_type=jnp.float32)
    z = jnp.dot(ph_ref[...], ws_ref[...], preferred_element_type=jnp.float32)
    gate_ref[...] = jax.nn.sigmoid(z + b_ref[...])


def _tc_dense(x, prev_h, wn, wl, ws, b2d):
    grid = (N // BLK,)
    row_spec = pl.BlockSpec((BLK, D), lambda i: (i, 0))
    w_spec = pl.BlockSpec((D, D), lambda i: (0, 0))
    b_spec = pl.BlockSpec((1, D), lambda i: (0, 0))
    out_sds = jax.ShapeDtypeStruct((N, D), jnp.float32)
    return pl.pallas_call(
        _dense_body,
        grid=grid,
        in_specs=[row_spec, row_spec, w_spec, w_spec, w_spec, b_spec],
        out_specs=[row_spec, row_spec, row_spec],
        out_shape=[out_sds, out_sds, out_sds],
    )(x, prev_h, wn, wl, ws, b2d)


def _embw_body(e_ref, w_ref, o_ref):
    o_ref[...] = jnp.dot(e_ref[...], w_ref[...],
                         preferred_element_type=jnp.float32)


def _tc_embw(emb_rel, wn):
    return pl.pallas_call(
        _embw_body,
        out_shape=jax.ShapeDtypeStruct((R, D), jnp.float32),
    )(emb_rel, wn)


def _sc_edge_body(xw_hbm, embw_hbm, srcIlo, srcIhi, typI, dstI, zeros_hbm,
                  out_hbm, acc, embw_sp, *bufs):
    c = lax.axis_index("c")
    s = lax.axis_index("s")
    wid = c * NS + s
    srclo = bufs[0:NIDX]
    srchi = bufs[NIDX:2 * NIDX]
    typb = bufs[2 * NIDX:3 * NIDX]
    dstb = bufs[3 * NIDX:4 * NIDX]
    xbufs = bufs[4 * NIDX:4 * NIDX + NBUF]
    rbufs = bufs[4 * NIDX + NBUF:4 * NIDX + 2 * NBUF]
    semoff = 4 * NIDX + 2 * NBUF
    isem = bufs[semoff:semoff + NIDX]
    gsem = bufs[semoff + NIDX:semoff + NIDX + NBUF]
    ssem = bufs[semoff + NIDX + NBUF:semoff + NIDX + 2 * NBUF]
    rsem = bufs[semoff + NIDX + 2 * NBUF:semoff + NIDX + 3 * NBUF]

    pltpu.sync_copy(zeros_hbm, acc.at[pl.ds(s * ROWS_PER_TILE, ROWS_PER_TILE)])

    @pl.when(s == 0)
    def _():
        pltpu.sync_copy(embw_hbm, embw_sp)

    def i_issue(i, q):
        pltpu.async_copy(srcIlo.at[wid, i], srclo[q], isem[q])
        pltpu.async_copy(srcIhi.at[wid, i], srchi[q], isem[q])
        pltpu.async_copy(typI.at[wid, i], typb[q], isem[q])
        pltpu.async_copy(dstI.at[wid, i], dstb[q], isem[q])

    def i_wait(q):
        pltpu.make_async_copy(srcIlo.at[0, 0], srclo[q], isem[q]).wait()
        pltpu.make_async_copy(srcIhi.at[0, 0], srchi[q], isem[q]).wait()
        pltpu.make_async_copy(typI.at[0, 0], typb[q], isem[q]).wait()
        pltpu.make_async_copy(dstI.at[0, 0], dstb[q], isem[q]).wait()

    def g_issue(q, b):
        pltpu.async_copy(xw_hbm.at[srclo[q]],
                         xbufs[b].at[pl.ds(0, KH)], gsem[b])
        pltpu.async_copy(xw_hbm.at[srchi[q]],
                         xbufs[b].at[pl.ds(KH, KH)], gsem[b])
        pltpu.async_copy(embw_sp.at[typb[q]], rbufs[b], rsem[b])

    def g_wait(b):
        pltpu.make_async_copy(xw_hbm.at[srclo[0]],
                              xbufs[b].at[pl.ds(0, KH)], gsem[b]).wait()
        pltpu.make_async_copy(xw_hbm.at[srchi[0]],
                              xbufs[b].at[pl.ds(KH, KH)], gsem[b]).wait()
        pltpu.make_async_copy(embw_sp.at[typb[0]], rbufs[b], rsem[b]).wait()

    def s_issue(q, b):
        pltpu.async_copy(xbufs[b], acc.at[dstb[q]], ssem[b], add=True)
        pltpu.async_copy(rbufs[b], acc.at[dstb[q]], ssem[b], add=True)

    def s_wait(b):
        pltpu.make_async_copy(xbufs[b], acc.at[dstb[0]], ssem[b]).wait()
        pltpu.make_async_copy(rbufs[b], acc.at[dstb[0]], ssem[b]).wait()

    for i in range(6):
        i_issue(i, i)
    plsc.subcore_barrier()
    i_wait(0)
    g_issue(0, 0)
    i_wait(1)
    g_issue(1, 1)

    def outer(jo, carry):
        for u in range(NIDX):
            i = NIDX * jo + u
            b = u % NBUF
            bp = (b + 2) % NBUF
            q2 = (u + 2) % NIDX
            q6 = (u + 6) % NIDX
            g_wait(b)
            s_issue(u, b)

            @pl.when(i >= 2)
            def _():
                s_wait(bp)

            @pl.when(i <= CHUNKS - 3)
            def _():
                i_wait(q2)
                g_issue(q2, bp)

            @pl.when(i <= CHUNKS - 7)
            def _():
                i_issue(i + 6, q6)

        return carry

    lax.fori_loop(0, CHUNKS // NIDX, outer, 0)
    s_wait((CHUNKS - 2) % NBUF)
    s_wait((CHUNKS - 1) % NBUF)
    plsc.subcore_barrier()
    pltpu.sync_copy(acc.at[pl.ds(s * ROWS_PER_TILE, ROWS_PER_TILE)],
                    out_hbm.at[c, pl.ds(s * ROWS_PER_TILE, ROWS_PER_TILE)])


def _sc_edges(xw, embw, srcIlo, srcIhi, typI, dstI, zeros):
    mesh = plsc.VectorSubcoreMesh(core_axis_name="c", subcore_axis_name="s")
    fn = functools.partial(
        pl.kernel,
        mesh=mesh,
        out_type=jax.ShapeDtypeStruct((NC, NPAD, D), jnp.float32),
        scratch_types=[pltpu.VMEM_SHARED((NPAD, D), jnp.float32),
                       pltpu.VMEM_SHARED((R, D), jnp.float32)]
        + [pltpu.VMEM((KH,), jnp.int32)] * (2 * NIDX)
        + [pltpu.VMEM((K,), jnp.int32)] * (2 * NIDX)
        + [pltpu.VMEM((K, D), jnp.float32)] * (2 * NBUF)
        + [pltpu.SemaphoreType.DMA] * (NIDX + 3 * NBUF),
    )(_sc_edge_body)
    return fn(xw, embw, srcIlo, srcIhi, typI, dstI, zeros)


def _final_body(p_ref, norm_ref, loop_ref, gate_ref, prev_ref, o_ref):
    agg = p_ref[0] + p_ref[1]
    h = agg * norm_ref[...] + loop_ref[...]
    g = gate_ref[...]
    o_ref[...] = g * h + (1.0 - g) * prev_ref[...]


def _tc_final(partials, norm, loop_m, gate, prev_h):
    grid = (N // BLK,)
    row_spec = pl.BlockSpec((BLK, D), lambda i: (i, 0))
    p_spec = pl.BlockSpec((NC, BLK, D), lambda i: (0, i, 0))
    n_spec = pl.BlockSpec((BLK, 1), lambda i: (i, 0))
    return pl.pallas_call(
        _final_body,
        grid=grid,
        in_specs=[p_spec, n_spec, row_spec, row_spec, row_spec],
        out_specs=pl.BlockSpec((BLK, D), lambda i: (i, 0)),
        out_shape=jax.ShapeDtypeStruct((N, D), jnp.float32),
    )(partials, norm, loop_m, gate, prev_h)


def kernel(x, edge_index, edge_type, norm, prev_h, emb_rel,
           weight_neighbor, loop_weight, skip_connect_weight,
           skip_connect_bias):
    src = edge_index[0]
    dst = edge_index[1]
    b2d = skip_connect_bias.reshape(1, D)
    xw, loop_m, gate = _tc_dense(x, prev_h, weight_neighbor, loop_weight,
                                 skip_connect_weight, b2d)
    embw = _tc_embw(emb_rel, weight_neighbor)
    zeros = jnp.zeros((ROWS_PER_TILE, D), jnp.float32)
    pad = EPAD - E
    srcI = jnp.concatenate([src, jnp.zeros((pad,), jnp.int32)]
                           ).reshape(NW, CHUNKS, K)
    typI = jnp.concatenate([edge_type, jnp.zeros((pad,), jnp.int32)]
                           ).reshape(NW, CHUNKS, K)
    dstI = jnp.concatenate([dst, jnp.full((pad,), N, jnp.int32)]
                           ).reshape(NW, CHUNKS, K)
    srcIlo = srcI[:, :, :KH] + 0
    srcIhi = srcI[:, :, KH:] + 0
    partials = _sc_edges(xw, embw, srcIlo, srcIhi, typI, dstI, zeros)
    return _tc_final(partials[:, :N], norm, loop_m, gate, prev_h)


# speedup vs baseline: 1.1705x; 1.0824x over previous
import functools

import jax
import jax.numpy as jnp
from jax import lax
from jax.experimental import pallas as pl
from jax.experimental.pallas import tpu as pltpu
from jax.experimental.pallas import tpu_sc as plsc

N = 10000
E = 320000
D = 128
R = 200

NC = 2
NS = 16
NW = NC * NS
NPAD = 10240
ROWS_PER_TILE = NPAD // NS
K = 40
KH = K // 2
CHUNKS = 256
EDGES_PER_TILE = CHUNKS * K
EPAD = NW * EDGES_PER_TILE
NBUF = 4
NIDX = 8

BLK = 2000


def _dense_body(x_ref, ph_ref, wn_ref, wl_ref, ws_ref, b_ref,
                xw_ref, loop_ref, gate_ref):
    xb = x_ref[...]
    xw_ref[...] = jnp.dot(xb, wn_ref[...], preferred_element_type=jnp.float32)
    loop_ref[...] = jnp.dot(xb, wl_ref[...], preferred_element_type=jnp.float32)
    z = jnp.dot(ph_ref[...], ws_ref[...], preferred_element_type=jnp.float32)
    gate_ref[...] = jax.nn.sigmoid(z + b_ref[...])


def _tc_dense(x, prev_h, wn, wl, ws, b2d):
    grid = (N // BLK,)
    row_spec = pl.BlockSpec((BLK, D), lambda i: (i, 0))
    w_spec = pl.BlockSpec((D, D), lambda i: (0, 0))
    b_spec = pl.BlockSpec((1, D), lambda i: (0, 0))
    out_sds = jax.ShapeDtypeStruct((N, D), jnp.float32)
    return pl.pallas_call(
        _dense_body,
        grid=grid,
        in_specs=[row_spec, row_spec, w_spec, w_spec, w_spec, b_spec],
        out_specs=[row_spec, row_spec, row_spec],
        out_shape=[out_sds, out_sds, out_sds],
    )(x, prev_h, wn, wl, ws, b2d)


def _embw_body(e_ref, w_ref, o_ref):
    o_ref[...] = jnp.dot(e_ref[...], w_ref[...],
                         preferred_element_type=jnp.float32)


def _tc_embw(emb_rel, wn):
    return pl.pallas_call(
        _embw_body,
        out_shape=jax.ShapeDtypeStruct((R, D), jnp.float32),
    )(emb_rel, wn)


def _sc_edge_body(xw_hbm, embw_hbm, srcIlo, srcIhi, typI, dstI, zeros_hbm,
                  out_hbm, acc, embw_sp, *bufs):
    c = lax.axis_index("c")
    s = lax.axis_index("s")
    wid = c * NS + s
    srclo = bufs[0:NIDX]
    srchi = bufs[NIDX:2 * NIDX]
    typb = bufs[2 * NIDX:3 * NIDX]
    dstb = bufs[3 * NIDX:4 * NIDX]
    xbufs = bufs[4 * NIDX:4 * NIDX + NBUF]
    rbufs = bufs[4 * NIDX + NBUF:4 * NIDX + 2 * NBUF]
    semoff = 4 * NIDX + 2 * NBUF
    isem = bufs[semoff:semoff + NIDX]
    gsem = bufs[semoff + NIDX:semoff + NIDX + NBUF]
    ssem = bufs[semoff + NIDX + NBUF:semoff + NIDX + 2 * NBUF]
    rsem = bufs[semoff + NIDX + 2 * NBUF:semoff + NIDX + 3 * NBUF]

    pltpu.sync_copy(zeros_hbm, acc.at[pl.ds(s * ROWS_PER_TILE, ROWS_PER_TILE)])

    @pl.when(s == 0)
    def _():
        pltpu.sync_copy(embw_hbm, embw_sp)

    def i_issue(i, q):
        pltpu.async_copy(srcIlo.at[wid, i], srclo[q], isem[q])
        pltpu.async_copy(srcIhi.at[wid, i], srchi[q], isem[q])
        pltpu.async_copy(typI.at[wid, i], typb[q], isem[q])
        pltpu.async_copy(dstI.at[wid, i], dstb[q], isem[q])

    def i_wait(q):
        pltpu.make_async_copy(srcIlo.at[0, 0], srclo[q], isem[q]).wait()
        pltpu.make_async_copy(srcIhi.at[0, 0], srchi[q], isem[q]).wait()
        pltpu.make_async_copy(typI.at[0, 0], typb[q], isem[q]).wait()
        pltpu.make_async_copy(dstI.at[0, 0], dstb[q], isem[q]).wait()

    def g_issue(q, b):
        pltpu.async_copy(xw_hbm.at[srclo[q]],
                         xbufs[b].at[pl.ds(0, KH)], gsem[b])
        pltpu.async_copy(xw_hbm.at[srchi[q]],
                         xbufs[b].at[pl.ds(KH, KH)], gsem[b])
        pltpu.async_copy(embw_sp.at[typb[q]], rbufs[b], rsem[b])

    def g_wait(b):
        pltpu.make_async_copy(xw_hbm.at[srclo[0]],
                              xbufs[b].at[pl.ds(0, KH)], gsem[b]).wait()
        pltpu.make_async_copy(xw_hbm.at[srchi[0]],
                              xbufs[b].at[pl.ds(KH, KH)], gsem[b]).wait()
        pltpu.make_async_copy(embw_sp.at[typb[0]], rbufs[b], rsem[b]).wait()

    def merge(b):
        def row(r, carry):
            for l in range(D // 16):
                sl = pl.ds(16 * l, 16)
                xbufs[b][r, sl] = xbufs[b][r, sl] + rbufs[b][r, sl]
            return carry

        lax.fori_loop(0, K, row, 0)

    def s_issue(q, b):
        pltpu.async_copy(xbufs[b], acc.at[dstb[q]], ssem[b], add=True)

    def s_wait(b):
        pltpu.make_async_copy(xbufs[b], acc.at[dstb[0]], ssem[b]).wait()

    for i in range(6):
        i_issue(i, i)
    plsc.subcore_barrier()
    i_wait(0)
    g_issue(0, 0)
    i_wait(1)
    g_issue(1, 1)

    def outer(jo, carry):
        for u in range(NIDX):
            i = NIDX * jo + u
            b = u % NBUF
            bp = (b + 2) % NBUF
            q2 = (u + 2) % NIDX
            q6 = (u + 6) % NIDX
            g_wait(b)
            merge(b)
            s_issue(u, b)

            @pl.when(i >= 2)
            def _():
                s_wait(bp)

            @pl.when(i <= CHUNKS - 3)
            def _():
                i_wait(q2)
                g_issue(q2, bp)

            @pl.when(i <= CHUNKS - 7)
            def _():
                i_issue(i + 6, q6)

        return carry

    lax.fori_loop(0, CHUNKS // NIDX, outer, 0)
    s_wait((CHUNKS - 2) % NBUF)
    s_wait((CHUNKS - 1) % NBUF)
    plsc.subcore_barrier()
    pltpu.sync_copy(acc.at[pl.ds(s * ROWS_PER_TILE, ROWS_PER_TILE)],
                    out_hbm.at[c, pl.ds(s * ROWS_PER_TILE, ROWS_PER_TILE)])


def _sc_edges(xw, embw, srcIlo, srcIhi, typI, dstI, zeros):
    mesh = plsc.VectorSubcoreMesh(core_axis_name="c", subcore_axis_name="s")
    fn = functools.partial(
        pl.kernel,
        mesh=mesh,
        out_type=jax.ShapeDtypeStruct((NC, NPAD, D), jnp.float32),
        scratch_types=[pltpu.VMEM_SHARED((NPAD, D), jnp.float32),
                       pltpu.VMEM_SHARED((R, D), jnp.float32)]
        + [pltpu.VMEM((KH,), jnp.int32)] * (2 * NIDX)
        + [pltpu.VMEM((K,), jnp.int32)] * (2 * NIDX)
        + [pltpu.VMEM((K, D), jnp.float32)] * (2 * NBUF)
        + [pltpu.SemaphoreType.DMA] * (NIDX + 3 * NBUF),
    )(_sc_edge_body)
    return fn(xw, embw, srcIlo, srcIhi, typI, dstI, zeros)


def _final_body(p_ref, norm_ref, loop_ref, gate_ref, prev_ref, o_ref):
    agg = p_ref[0] + p_ref[1]
    h = agg * norm_ref[...] + loop_ref[...]
    g = gate_ref[...]
    o_ref[...] = g * h + (1.0 - g) * prev_ref[...]


def _tc_final(partials, norm, loop_m, gate, prev_h):
    grid = (N // BLK,)
    row_spec = pl.BlockSpec((BLK, D), lambda i: (i, 0))
    p_spec = pl.BlockSpec((NC, BLK, D), lambda i: (0, i, 0))
    n_spec = pl.BlockSpec((BLK, 1), lambda i: (i, 0))
    return pl.pallas_call(
        _final_body,
        grid=grid,
        in_specs=[p_spec, n_spec, row_spec, row_spec, row_spec],
        out_specs=pl.BlockSpec((BLK, D), lambda i: (i, 0)),
        out_shape=jax.ShapeDtypeStruct((N, D), jnp.float32),
    )(partials, norm, loop_m, gate, prev_h)


def kernel(x, edge_index, edge_type, norm, prev_h, emb_rel,
           weight_neighbor, loop_weight, skip_connect_weight,
           skip_connect_bias):
    src = edge_index[0]
    dst = edge_index[1]
    b2d = skip_connect_bias.reshape(1, D)
    xw, loop_m, gate = _tc_dense(x, prev_h, weight_neighbor, loop_weight,
                                 skip_connect_weight, b2d)
    embw = _tc_embw(emb_rel, weight_neighbor)
    zeros = jnp.zeros((ROWS_PER_TILE, D), jnp.float32)
    pad = EPAD - E
    srcI = jnp.concatenate([src, jnp.zeros((pad,), jnp.int32)]
                           ).reshape(NW, CHUNKS, K)
    typI = jnp.concatenate([edge_type, jnp.zeros((pad,), jnp.int32)]
                           ).reshape(NW, CHUNKS, K)
    dstI = jnp.concatenate([dst, jnp.full((pad,), N, jnp.int32)]
                           ).reshape(NW, CHUNKS, K)
    srcIlo = srcI[:, :, :KH] + 0
    srcIhi = srcI[:, :, KH:] + 0
    partials = _sc_edges(xw, embw, srcIlo, srcIhi, typI, dstI, zeros)
    return _tc_final(partials[:, :N], norm, loop_m, gate, prev_h)
